# baseline (speedup 1.0000x reference)
import functools

import jax
import jax.numpy as jnp
import numpy as np
from jax.experimental import pallas as pl
from jax.experimental.pallas import tpu as pltpu

_TOPK = 512
_SQRT_HALF = np.float32(np.sqrt(0.5))
_ONE_BITS = np.int32(0x3F800001)


def _fused_kernel(x_ref, w1t_ref, w2_ref, b1_ref, b2_ref, gb_ref,
                  out_ref, mg_ref,
                  acc, xres, gate, mg_acc,
                  *, B, T, D, KP, TB, NT):
    b = pl.program_id(0)
    p = pl.program_id(1)
    t = pl.program_id(2)
    k = min(_TOPK, D)

    @pl.when(p == 0)
    def _phase0():
        xblk = x_ref[0]

        @pl.when(t == 0)
        def _():
            acc[...] = jnp.zeros_like(acc)

        a = acc[...]
        for j in range(TB // 8):
            a = a + xblk[8 * j:8 * j + 8, :]
        acc[...] = a

        xres[pl.ds(t * TB, TB), :] = xblk.astype(jnp.bfloat16)

        @pl.when(t == NT - 1)
        def _gate_and_topk():
            a = acc[...]
            b4 = a[:4, :] + a[4:, :]
            b2 = b4[:2, :] + b4[2:, :]
            ctx = (b2[0:1, :] + b2[1:2, :]) * np.float32(1.0 / T)
            row = jax.lax.broadcasted_iota(jnp.int32, (8, D), 0)
            ctx8 = jnp.where(row == 0, jnp.broadcast_to(ctx, (8, D)),
                             jnp.float32(0.0))
            cb = ctx8.astype(jnp.bfloat16)
            w1b = w1t_ref[...].astype(jnp.bfloat16)
            hpre = jax.lax.dot_general(
                cb, w1b, (((1,), (1,)), ((), ())),
                preferred_element_type=jnp.float32)
            hpre = hpre + b1_ref[...]
            h = 0.5 * hpre * (1.0 - jax.lax.erf(-hpre * _SQRT_HALF))
            hb = h.astype(jnp.bfloat16)
            w2b = w2_ref[...].astype(jnp.bfloat16)
            gp = jax.lax.dot_general(
                hb, w2b, (((1,), (0,)), ((), ())),
                preferred_element_type=jnp.float32)
            gp = (gp + b2_ref[...]) + gb_ref[...]
            g = jax.nn.sigmoid(gp[0:1, :])

            gr = jnp.reshape(g, (8, D // 8))
            gi = jax.lax.bitcast_convert_type(gr, jnp.int32)

            def _count_ge(thr):
                return jnp.sum((gi >= thr).astype(jnp.int32))

            def _vbody(_, lohi):
                lo, hi = lohi
                mid = (lo + hi) // 2
                ge = _count_ge(mid) >= k
                return (jnp.where(ge, mid, lo), jnp.where(ge, hi, mid))

            lo, hi = jax.lax.fori_loop(
                0, 31, _vbody, (jnp.int32(0), _ONE_BITS))
            c_gt = jnp.sum((gi > lo).astype(jnp.int32))
            r = k - c_gt
            eq = gi == lo
            idx = (jax.lax.broadcasted_iota(jnp.int32, (8, D // 8), 0)
                   * (D // 8)
                   + jax.lax.broadcasted_iota(jnp.int32, (8, D // 8), 1))

            def _ibody(_, lohi):
                lo_i, hi_i = lohi
                mid = (lo_i + hi_i) // 2
                cnt = jnp.sum(jnp.where(eq & (idx <= mid), 1, 0))
                ge = cnt >= r
                return (jnp.where(ge, lo_i, mid), jnp.where(ge, mid, hi_i))

            _, isel = jax.lax.fori_loop(
                0, 13, _ibody, (jnp.int32(-1), jnp.int32(D - 1)))
            mask = (gi > lo) | (eq & (idx <= isel))
            gm = jnp.where(mask, gr, jnp.float32(0.0))
            gate[...] = jnp.reshape(gm, (1, D))

            s = jnp.sum(gm)

            @pl.when(b == 0)
            def _():
                mg_acc[0] = s

            @pl.when(b > 0)
            def _():
                mg_acc[0] = mg_acc[0] + s

    @pl.when(p == 1)
    def _phase1():
        xb = xres[pl.ds(t * TB, TB), :].astype(jnp.float32)
        out_ref[0] = xb * gate[...]

        @pl.when((b == B - 1) & (t == NT - 1))
        def _():
            mg_ref[0, 0] = mg_acc[0] * np.float32(1.0 / (B * D))


def kernel(x, W1, b1, W2, b2, g_bias):
    B, T, D = x.shape
    KD = W1.shape[1]
    KP = 128
    TB = 128
    NT = T // TB

    W1t = jnp.zeros((KP, D), jnp.float32).at[:KD, :].set(W1.T)
    W2p = jnp.zeros((KP, D), jnp.float32).at[:KD, :].set(W2)
    b1p = jnp.zeros((1, KP), jnp.float32).at[0, :KD].set(b1)
    b2r = b2.reshape(1, D)
    gbr = g_bias.reshape(1, D)

    fn = functools.partial(_fused_kernel, B=B, T=T, D=D, KP=KP, TB=TB, NT=NT)
    x_out, mg = pl.pallas_call(
        fn,
        grid=(B, 2, NT),
        in_specs=[
            pl.BlockSpec((1, TB, D),
                         lambda b, p, t: (b, t * (1 - p) + (NT - 1) * p, 0)),
            pl.BlockSpec((KP, D), lambda b, p, t: (0, 0)),
            pl.BlockSpec((KP, D), lambda b, p, t: (0, 0)),
            pl.BlockSpec((1, KP), lambda b, p, t: (0, 0)),
            pl.BlockSpec((1, D), lambda b, p, t: (0, 0)),
            pl.BlockSpec((1, D), lambda b, p, t: (0, 0)),
        ],
        out_specs=[
            pl.BlockSpec((1, TB, D),
                         lambda b, p, t: (b, t * p, 0)),
            pl.BlockSpec((1, 1), lambda b, p, t: (0, 0),
                         memory_space=pltpu.SMEM),
        ],
        out_shape=[
            jax.ShapeDtypeStruct((B, T, D), jnp.float32),
            jax.ShapeDtypeStruct((1, 1), jnp.float32),
        ],
        scratch_shapes=[
            pltpu.VMEM((8, D), jnp.float32),
            pltpu.VMEM((T, D), jnp.bfloat16),
            pltpu.VMEM((1, D), jnp.float32),
            pltpu.SMEM((1,), jnp.float32),
        ],
        compiler_params=pltpu.CompilerParams(
            dimension_semantics=("arbitrary", "arbitrary", "arbitrary"),
        ),
    )(x, W1t, W2p, b1p, b2r, gbr)

    topk_frac = jnp.float32(min(_TOPK, D)) / jnp.float32(D)
    return (x_out, mg[0, 0], topk_frac)


# speedup vs baseline: 1.1821x; 1.1821x over previous
import functools

import jax
import jax.numpy as jnp
import numpy as np
from jax.experimental import pallas as pl
from jax.experimental.pallas import tpu as pltpu

_TOPK = 512
_SQRT_HALF = np.float32(np.sqrt(0.5))
_ONE_BITS = np.int32(0x3F800001)


def _fused_kernel(x_ref, w1t_ref, w2_ref, b1_ref, b2_ref, gb_ref,
                  out_ref, mg_ref,
                  acc, xres, gate, mg_acc,
                  *, B, T, D, KP, TB, NT):
    b = pl.program_id(0)
    p = pl.program_id(1)
    t = pl.program_id(2)
    k = min(_TOPK, D)

    @pl.when(p == 0)
    def _phase0():
        xblk = x_ref[0]

        @pl.when(t == 0)
        def _():
            acc[...] = jnp.zeros_like(acc)

        a = acc[...]
        for j in range(TB // 8):
            a = a + xblk[8 * j:8 * j + 8, :]
        acc[...] = a

        xres[pl.ds(t * TB, TB), :] = xblk.astype(jnp.bfloat16)

        @pl.when(t == NT - 1)
        def _gate_and_topk():
            a = acc[...]
            b4 = a[:4, :] + a[4:, :]
            b2 = b4[:2, :] + b4[2:, :]
            ctx = (b2[0:1, :] + b2[1:2, :]) * np.float32(1.0 / T)
            row = jax.lax.broadcasted_iota(jnp.int32, (8, D), 0)
            ctx8 = jnp.where(row == 0, jnp.broadcast_to(ctx, (8, D)),
                             jnp.float32(0.0))
            cb = ctx8.astype(jnp.bfloat16)
            w1b = w1t_ref[...]
            hpre = jax.lax.dot_general(
                cb, w1b, (((1,), (1,)), ((), ())),
                preferred_element_type=jnp.float32)
            hpre = hpre + b1_ref[...]
            h = 0.5 * hpre * (1.0 - jax.lax.erf(-hpre * _SQRT_HALF))
            hb = h.astype(jnp.bfloat16)
            w2b = w2_ref[...]
            gp = jax.lax.dot_general(
                hb, w2b, (((1,), (0,)), ((), ())),
                preferred_element_type=jnp.float32)
            gp = (gp + b2_ref[...]) + gb_ref[...]
            g = jax.nn.sigmoid(gp[0:1, :])

            gr = jnp.reshape(g, (8, D // 8))
            gi = jax.lax.bitcast_convert_type(gr, jnp.int32)

            def _count_ge(thr):
                return jnp.sum((gi >= thr).astype(jnp.int32))

            def _vbody(_, lohi):
                lo, hi = lohi
                mid = (lo + hi) // 2
                ge = _count_ge(mid) >= k
                return (jnp.where(ge, mid, lo), jnp.where(ge, hi, mid))

            lo, hi = jax.lax.fori_loop(
                0, 31, _vbody, (jnp.int32(0), _ONE_BITS))
            c_gt = jnp.sum((gi > lo).astype(jnp.int32))
            r = k - c_gt
            eq = gi == lo
            idx = (jax.lax.broadcasted_iota(jnp.int32, (8, D // 8), 0)
                   * (D // 8)
                   + jax.lax.broadcasted_iota(jnp.int32, (8, D // 8), 1))

            def _ibody(_, lohi):
                lo_i, hi_i = lohi
                mid = (lo_i + hi_i) // 2
                cnt = jnp.sum(jnp.where(eq & (idx <= mid), 1, 0))
                ge = cnt >= r
                return (jnp.where(ge, lo_i, mid), jnp.where(ge, mid, hi_i))

            _, isel = jax.lax.fori_loop(
                0, 13, _ibody, (jnp.int32(-1), jnp.int32(D - 1)))
            mask = (gi > lo) | (eq & (idx <= isel))
            gm = jnp.where(mask, gr, jnp.float32(0.0))
            gate[...] = jnp.reshape(gm, (1, D))

            s = jnp.sum(gm)

            @pl.when(b == 0)
            def _():
                mg_acc[0] = s

            @pl.when(b > 0)
            def _():
                mg_acc[0] = mg_acc[0] + s

    @pl.when(p == 1)
    def _phase1():
        xb = xres[pl.ds(t * TB, TB), :].astype(jnp.float32)
        out_ref[0] = xb * gate[...]

        @pl.when((b == B - 1) & (t == NT - 1))
        def _():
            mg_ref[0, 0] = mg_acc[0] * np.float32(1.0 / (B * D))


def kernel(x, W1, b1, W2, b2, g_bias):
    B, T, D = x.shape
    KD = W1.shape[1]
    KP = 128
    TB = 512
    NT = T // TB

    W1t = jnp.zeros((KP, D), jnp.bfloat16).at[:KD, :].set(
        W1.T.astype(jnp.bfloat16))
    W2p = jnp.zeros((KP, D), jnp.bfloat16).at[:KD, :].set(
        W2.astype(jnp.bfloat16))
    b1p = jnp.zeros((1, KP), jnp.float32).at[0, :KD].set(b1)
    b2r = b2.reshape(1, D)
    gbr = g_bias.reshape(1, D)

    fn = functools.partial(_fused_kernel, B=B, T=T, D=D, KP=KP, TB=TB, NT=NT)
    x_out, mg = pl.pallas_call(
        fn,
        grid=(B, 2, NT),
        in_specs=[
            pl.BlockSpec((1, TB, D),
                         lambda b, p, t: (b, t * (1 - p) + (NT - 1) * p, 0)),
            pl.BlockSpec((KP, D), lambda b, p, t: (0, 0)),
            pl.BlockSpec((KP, D), lambda b, p, t: (0, 0)),
            pl.BlockSpec((1, KP), lambda b, p, t: (0, 0)),
            pl.BlockSpec((1, D), lambda b, p, t: (0, 0)),
            pl.BlockSpec((1, D), lambda b, p, t: (0, 0)),
        ],
        out_specs=[
            pl.BlockSpec((1, TB, D),
                         lambda b, p, t: (b, t * p, 0)),
            pl.BlockSpec((1, 1), lambda b, p, t: (0, 0),
                         memory_space=pltpu.SMEM),
        ],
        out_shape=[
            jax.ShapeDtypeStruct((B, T, D), jnp.float32),
            jax.ShapeDtypeStruct((1, 1), jnp.float32),
        ],
        scratch_shapes=[
            pltpu.VMEM((8, D), jnp.float32),
            pltpu.VMEM((T, D), jnp.bfloat16),
            pltpu.VMEM((1, D), jnp.float32),
            pltpu.SMEM((1,), jnp.float32),
        ],
        compiler_params=pltpu.CompilerParams(
            dimension_semantics=("arbitrary", "arbitrary", "arbitrary"),
        ),
    )(x, W1t, W2p, b1p, b2r, gbr)

    topk_frac = jnp.float32(min(_TOPK, D)) / jnp.float32(D)
    return (x_out, mg[0, 0], topk_frac)


# speedup vs baseline: 1.2527x; 1.0597x over previous
import functools

import jax
import jax.numpy as jnp
import numpy as np
from jax.experimental import pallas as pl
from jax.experimental.pallas import tpu as pltpu

_TOPK = 512
_SQRT_HALF = np.float32(np.sqrt(0.5))
_ONE_BITS = np.int32(0x3F800001)


def _fused_kernel(x_ref, w1t_ref, w2_ref, b1_ref, b2_ref, gb_ref,
                  out_ref, mg_ref,
                  acc, xres, gate, mg_acc,
                  *, B, T, D, KP, TB, NT):
    b = pl.program_id(0)
    t = pl.program_id(1)
    k = min(_TOPK, D)
    slot = jax.lax.rem(b, 2)

    @pl.when(b < B)
    def _phase0():
        xblk = x_ref[0]

        @pl.when(t == 0)
        def _():
            acc[...] = jnp.zeros_like(acc)

        a = acc[...]
        for j in range(TB // 8):
            a = a + xblk[8 * j:8 * j + 8, :]
        acc[...] = a

        xres[pl.ds(slot * T + t * TB, TB), :] = xblk.astype(jnp.bfloat16)

        @pl.when(t == NT - 1)
        def _gate_and_topk():
            a = acc[...]
            b4 = a[:4, :] + a[4:, :]
            b2 = b4[:2, :] + b4[2:, :]
            ctx = (b2[0:1, :] + b2[1:2, :]) * np.float32(1.0 / T)
            row = jax.lax.broadcasted_iota(jnp.int32, (8, D), 0)
            ctx8 = jnp.where(row == 0, jnp.broadcast_to(ctx, (8, D)),
                             jnp.float32(0.0))
            cb = ctx8.astype(jnp.bfloat16)
            w1b = w1t_ref[...]
            hpre = jax.lax.dot_general(
                cb, w1b, (((1,), (1,)), ((), ())),
                preferred_element_type=jnp.float32)
            hpre = hpre + b1_ref[...]
            h = 0.5 * hpre * (1.0 - jax.lax.erf(-hpre * _SQRT_HALF))
            hb = h.astype(jnp.bfloat16)
            w2b = w2_ref[...]
            gp = jax.lax.dot_general(
                hb, w2b, (((1,), (0,)), ((), ())),
                preferred_element_type=jnp.float32)
            gp = (gp + b2_ref[...]) + gb_ref[...]
            g = jax.nn.sigmoid(gp[0:1, :])

            gr = jnp.reshape(g, (8, D // 8))
            gi = jax.lax.bitcast_convert_type(gr, jnp.int32)

            def _vbody(_, lohi):
                lo, hi = lohi
                mid = (lo + hi) // 2
                ge = jnp.sum((gi >= mid).astype(jnp.int32)) >= k
                return (jnp.where(ge, mid, lo), jnp.where(ge, hi, mid))

            lo, hi = jax.lax.fori_loop(
                0, 31, _vbody, (jnp.int32(0), _ONE_BITS))
            c_gt = jnp.sum((gi > lo).astype(jnp.int32))
            r = k - c_gt
            eq = gi == lo
            idx = (jax.lax.broadcasted_iota(jnp.int32, (8, D // 8), 0)
                   * (D // 8)
                   + jax.lax.broadcasted_iota(jnp.int32, (8, D // 8), 1))

            def _ibody(_, lohi):
                lo_i, hi_i = lohi
                mid = (lo_i + hi_i) // 2
                cnt = jnp.sum(jnp.where(eq & (idx <= mid), 1, 0))
                ge = cnt >= r
                return (jnp.where(ge, lo_i, mid), jnp.where(ge, mid, hi_i))

            _, isel = jax.lax.fori_loop(
                0, 13, _ibody, (jnp.int32(-1), jnp.int32(D - 1)))
            mask = (gi > lo) | (eq & (idx <= isel))
            gm = jnp.where(mask, gr, jnp.float32(0.0))
            gate[pl.ds(slot, 1), :] = jnp.reshape(gm, (1, D))

            s = jnp.sum(gm)

            @pl.when(b == 0)
            def _():
                mg_acc[0] = s

            @pl.when(b > 0)
            def _():
                mg_acc[0] = mg_acc[0] + s

    @pl.when(b >= 1)
    def _phase1():
        pslot = 1 - slot
        xb = xres[pl.ds(pslot * T + t * TB, TB), :].astype(jnp.float32)
        out_ref[0] = xb * gate[pl.ds(pslot, 1), :]

        @pl.when((b == B) & (t == NT - 1))
        def _():
            mg_ref[0, 0] = mg_acc[0] * np.float32(1.0 / (B * D))


def kernel(x, W1, b1, W2, b2, g_bias):
    B, T, D = x.shape
    KD = W1.shape[1]
    KP = 128
    TB = 256
    NT = T // TB

    W1t = jnp.zeros((KP, D), jnp.bfloat16).at[:KD, :].set(
        W1.T.astype(jnp.bfloat16))
    W2p = jnp.zeros((KP, D), jnp.bfloat16).at[:KD, :].set(
        W2.astype(jnp.bfloat16))
    b1p = jnp.zeros((1, KP), jnp.float32).at[0, :KD].set(b1)
    b2r = b2.reshape(1, D)
    gbr = g_bias.reshape(1, D)

    fn = functools.partial(_fused_kernel, B=B, T=T, D=D, KP=KP, TB=TB, NT=NT)
    x_out, mg = pl.pallas_call(
        fn,
        grid=(B + 1, NT),
        in_specs=[
            pl.BlockSpec(
                (1, TB, D),
                lambda b, t: (jnp.minimum(b, B - 1),
                              jnp.where(b < B, t, NT - 1), 0)),
            pl.BlockSpec((KP, D), lambda b, t: (0, 0)),
            pl.BlockSpec((KP, D), lambda b, t: (0, 0)),
            pl.BlockSpec((1, KP), lambda b, t: (0, 0)),
            pl.BlockSpec((1, D), lambda b, t: (0, 0)),
            pl.BlockSpec((1, D), lambda b, t: (0, 0)),
        ],
        out_specs=[
            pl.BlockSpec(
                (1, TB, D),
                lambda b, t: (jnp.maximum(b - 1, 0),
                              jnp.where(b >= 1, t, 0), 0)),
            pl.BlockSpec((1, 1), lambda b, t: (0, 0),
                         memory_space=pltpu.SMEM),
        ],
        out_shape=[
            jax.ShapeDtypeStruct((B, T, D), jnp.float32),
            jax.ShapeDtypeStruct((1, 1), jnp.float32),
        ],
        scratch_shapes=[
            pltpu.VMEM((8, D), jnp.float32),
            pltpu.VMEM((2 * T, D), jnp.bfloat16),
            pltpu.VMEM((2, D), jnp.float32),
            pltpu.SMEM((1,), jnp.float32),
        ],
        compiler_params=pltpu.CompilerParams(
            dimension_semantics=("arbitrary", "arbitrary"),
        ),
    )(x, W1t, W2p, b1p, b2r, gbr)

    topk_frac = jnp.float32(min(_TOPK, D)) / jnp.float32(D)
    return (x_out, mg[0, 0], topk_frac)


# speedup vs baseline: 1.3160x; 1.0505x over previous
import functools

import jax
import jax.numpy as jnp
import numpy as np
from jax.experimental import pallas as pl
from jax.experimental.pallas import tpu as pltpu

_TOPK = 512
_SQRT_HALF = np.float32(np.sqrt(0.5))
_ONE_BITS = np.int32(0x3F800001)


def _fused_kernel(x_ref, w1t_ref, w2_ref, b1_ref, b2_ref, gb_ref,
                  out_ref, mg_ref,
                  acc, xres, gate, mg_acc,
                  *, B, T, D, KP, TB, NT):
    b = pl.program_id(0)
    t = pl.program_id(1)
    k = min(_TOPK, D)
    slot = jax.lax.rem(b, 2)

    @pl.when(b >= 1)
    def _phase1():
        pslot = 1 - slot
        xb = xres[pl.ds(t * TB, TB), :].astype(jnp.float32)
        out_ref[0] = xb * gate[pl.ds(pslot, 1), :]

        @pl.when((b == B) & (t == NT - 1))
        def _():
            mg_ref[0, 0] = mg_acc[0] * np.float32(1.0 / (B * D))

    @pl.when(b < B)
    def _phase0():
        xblk = x_ref[0]

        @pl.when(t == 0)
        def _():
            acc[...] = jnp.zeros_like(acc)

        a = acc[...]
        for j in range(TB // 8):
            a = a + xblk[8 * j:8 * j + 8, :]
        acc[...] = a

        xres[pl.ds(t * TB, TB), :] = xblk.astype(jnp.bfloat16)

        @pl.when(t == NT - 1)
        def _gate_and_topk():
            a = acc[...]
            b4 = a[:4, :] + a[4:, :]
            b2 = b4[:2, :] + b4[2:, :]
            ctx = (b2[0:1, :] + b2[1:2, :]) * np.float32(1.0 / T)
            row = jax.lax.broadcasted_iota(jnp.int32, (8, D), 0)
            ctx8 = jnp.where(row == 0, jnp.broadcast_to(ctx, (8, D)),
                             jnp.float32(0.0))
            cb = ctx8.astype(jnp.bfloat16)
            w1b = w1t_ref[...]
            hpre = jax.lax.dot_general(
                cb, w1b, (((1,), (1,)), ((), ())),
                preferred_element_type=jnp.float32)
            hpre = hpre + b1_ref[...]
            h = 0.5 * hpre * (1.0 - jax.lax.erf(-hpre * _SQRT_HALF))
            hb = h.astype(jnp.bfloat16)
            w2b = w2_ref[...]
            gp = jax.lax.dot_general(
                hb, w2b, (((1,), (0,)), ((), ())),
                preferred_element_type=jnp.float32)
            gp = (gp + b2_ref[...]) + gb_ref[...]
            g = jax.nn.sigmoid(gp[0:1, :])

            gr = jnp.reshape(g, (8, D // 8))
            gi = jax.lax.bitcast_convert_type(gr, jnp.int32)

            def _vbody(_, lohi):
                lo, hi = lohi
                mid = (lo + hi) // 2
                ge = jnp.sum((gi >= mid).astype(jnp.int32)) >= k
                return (jnp.where(ge, mid, lo), jnp.where(ge, hi, mid))

            lo, hi = jax.lax.fori_loop(
                0, 31, _vbody, (jnp.int32(0), _ONE_BITS))
            c_gt = jnp.sum((gi > lo).astype(jnp.int32))
            r = k - c_gt
            eq = gi == lo
            idx = (jax.lax.broadcasted_iota(jnp.int32, (8, D // 8), 0)
                   * (D // 8)
                   + jax.lax.broadcasted_iota(jnp.int32, (8, D // 8), 1))

            def _ibody(_, lohi):
                lo_i, hi_i = lohi
                mid = (lo_i + hi_i) // 2
                cnt = jnp.sum(jnp.where(eq & (idx <= mid), 1, 0))
                ge = cnt >= r
                return (jnp.where(ge, lo_i, mid), jnp.where(ge, mid, hi_i))

            _, isel = jax.lax.fori_loop(
                0, 13, _ibody, (jnp.int32(-1), jnp.int32(D - 1)))
            mask = (gi > lo) | (eq & (idx <= isel))
            gm = jnp.where(mask, gr, jnp.float32(0.0))
            gate[pl.ds(slot, 1), :] = jnp.reshape(gm, (1, D))

            s = jnp.sum(gm)

            @pl.when(b == 0)
            def _():
                mg_acc[0] = s

            @pl.when(b > 0)
            def _():
                mg_acc[0] = mg_acc[0] + s

def kernel(x, W1, b1, W2, b2, g_bias):
    B, T, D = x.shape
    KD = W1.shape[1]
    KP = 128
    TB = 512
    NT = T // TB

    W1t = jnp.zeros((KP, D), jnp.bfloat16).at[:KD, :].set(
        W1.T.astype(jnp.bfloat16))
    W2p = jnp.zeros((KP, D), jnp.bfloat16).at[:KD, :].set(
        W2.astype(jnp.bfloat16))
    b1p = jnp.zeros((1, KP), jnp.float32).at[0, :KD].set(b1)
    b2r = b2.reshape(1, D)
    gbr = g_bias.reshape(1, D)

    fn = functools.partial(_fused_kernel, B=B, T=T, D=D, KP=KP, TB=TB, NT=NT)
    x_out, mg = pl.pallas_call(
        fn,
        grid=(B + 1, NT),
        in_specs=[
            pl.BlockSpec(
                (1, TB, D),
                lambda b, t: (jnp.minimum(b, B - 1),
                              jnp.where(b < B, t, NT - 1), 0)),
            pl.BlockSpec((KP, D), lambda b, t: (0, 0)),
            pl.BlockSpec((KP, D), lambda b, t: (0, 0)),
            pl.BlockSpec((1, KP), lambda b, t: (0, 0)),
            pl.BlockSpec((1, D), lambda b, t: (0, 0)),
            pl.BlockSpec((1, D), lambda b, t: (0, 0)),
        ],
        out_specs=[
            pl.BlockSpec(
                (1, TB, D),
                lambda b, t: (jnp.maximum(b - 1, 0),
                              jnp.where(b >= 1, t, 0), 0)),
            pl.BlockSpec((1, 1), lambda b, t: (0, 0),
                         memory_space=pltpu.SMEM),
        ],
        out_shape=[
            jax.ShapeDtypeStruct((B, T, D), jnp.float32),
            jax.ShapeDtypeStruct((1, 1), jnp.float32),
        ],
        scratch_shapes=[
            pltpu.VMEM((8, D), jnp.float32),
            pltpu.VMEM((T, D), jnp.bfloat16),
            pltpu.VMEM((2, D), jnp.float32),
            pltpu.SMEM((1,), jnp.float32),
        ],
        compiler_params=pltpu.CompilerParams(
            dimension_semantics=("arbitrary", "arbitrary"),
        ),
    )(x, W1t, W2p, b1p, b2r, gbr)

    topk_frac = jnp.float32(min(_TOPK, D)) / jnp.float32(D)
    return (x_out, mg[0, 0], topk_frac)
